# baseline (speedup 1.0000x reference)
import jax
import jax.numpy as jnp
from jax.experimental import pallas as pl
from jax.experimental.pallas import tpu as pltpu

D = 768
S = 8192
H = 10
W = 32
BLOCK_S = 1024

_SHIFTS = tuple(sorted({20 + j - 2 * min(j, 9) for j in range(12)}))
_DESTS = {s: tuple(j for j in range(12) if 20 + j - 2 * min(j, 9) == s)
          for s in _SHIFTS}


def _b16(x):
    return x.astype(jnp.bfloat16)


def _f32(x):
    return x.astype(jnp.float32)


def _head_to_val_lanes(x):
    lane = jax.lax.broadcasted_iota(jnp.int32, x.shape, 1)
    out = x
    for s in _SHIFTS:
        rolled = pltpu.roll(x, s, 1)
        dmask = jnp.zeros(x.shape, dtype=jnp.int32)
        for j in _DESTS[s]:
            dmask = jnp.maximum(dmask, (lane == 20 + j).astype(jnp.int32))
        out = jnp.where(dmask > 0, rolled, out)
    return out


def _body(mem_ref, q2d_ref, wqf_ref, wall_ref, bqf_ref,
          vals_ref, bs_ref, bi_ref,
          qm_s, m_s, idx_s, v_s):
    step = pl.program_id(0)
    nsteps = pl.num_programs(0)

    @pl.when(step == 0)
    def _init():
        qrow = jax.lax.dot_general(
            _b16(q2d_ref[:]), _b16(wqf_ref[:]), (((1,), (1,)), ((), ())),
            preferred_element_type=jnp.float32) + bqf_ref[:]
        qrow32 = _f32(_b16(jnp.concatenate(
            [qrow, jnp.zeros((1, W - 2 * H), jnp.float32)], axis=1)))
        qfull = jnp.broadcast_to(qrow32, (W, W))
        rollc = pltpu.roll(qfull, W - 1, 1)
        rr = jax.lax.broadcasted_iota(jnp.int32, (W, W), 0)
        cc = jax.lax.broadcasted_iota(jnp.int32, (W, W), 1)
        head = rr < 2 * H
        even_diag = jnp.logical_and(jnp.logical_and(rr == cc, rr % 2 == 0), head)
        odd_sub = jnp.logical_and(jnp.logical_and(cc == rr - 1, rr % 2 == 1), head)
        qm_s[:] = _b16(jnp.where(even_diag, qfull, 0.0)
                       + jnp.where(odd_sub, rollc, 0.0))
        m_s[:] = jnp.full((1, W), -jnp.inf, dtype=jnp.float32)
        idx_s[:] = jnp.zeros((1, W), dtype=jnp.int32)
        v_s[:] = jnp.zeros((1, W), dtype=jnp.float32)

    scat = jax.lax.dot_general(mem_ref[:], wall_ref[:], (((1,), (1,)), ((), ())),
                               preferred_element_type=jnp.float32)
    scores = jax.lax.dot_general(
        scat, _f32(qm_s[:]), (((1,), (0,)), ((), ())),
        preferred_element_type=jnp.float32)

    m = jnp.max(scores, axis=0, keepdims=True)
    ii = jax.lax.broadcasted_iota(jnp.int32, scores.shape, 0)
    li = jnp.min(jnp.where(scores == m, ii, BLOCK_S), axis=0, keepdims=True)
    li_all = _head_to_val_lanes(li)
    sel = jnp.where(ii == li_all, scat, 0.0)
    v_cand = jnp.sum(sel, axis=0, keepdims=True)

    upd = m > m_s[:]
    updv = _head_to_val_lanes(upd.astype(jnp.int32)) > 0
    m_s[:] = jnp.where(upd, m, m_s[:])
    idx_s[:] = jnp.where(upd, li + step * BLOCK_S, idx_s[:])
    v_s[:] = jnp.where(updv, v_cand, v_s[:])

    @pl.when(step == nsteps - 1)
    def _fin():
        vals_ref[:] = v_s[:]
        bs_ref[:] = m_s[:]
        bi_ref[:] = idx_s[:]


def kernel(query_emb, memory_embs, WQ, bQ, WK, WV_small, WV_call):
    q2d = query_emb.reshape(1, D)
    WALL = jnp.concatenate(
        [WK.reshape(2 * H, D), WV_small.reshape(9, D), WV_call], axis=0)
    WQf = WQ.reshape(2 * H, D)
    bQf = bQ.reshape(1, 2 * H)

    nsteps = S // BLOCK_S
    full = lambda shape: pl.BlockSpec(shape, lambda i: (0, 0))
    vals, bs, bi = pl.pallas_call(
        _body,
        grid=(nsteps,),
        in_specs=[
            pl.BlockSpec((BLOCK_S, D), lambda i: (i, 0)),
            full((1, D)), full((2 * H, D)), full((W, D)), full((1, 2 * H)),
        ],
        out_specs=[full((1, W)), full((1, W)), full((1, W))],
        out_shape=[
            jax.ShapeDtypeStruct((1, W), jnp.float32),
            jax.ShapeDtypeStruct((1, W), jnp.float32),
            jax.ShapeDtypeStruct((1, W), jnp.int32),
        ],
        scratch_shapes=[
            pltpu.VMEM((W, W), jnp.bfloat16),
            pltpu.VMEM((1, W), jnp.float32),
            pltpu.VMEM((1, W), jnp.int32),
            pltpu.VMEM((1, W), jnp.float32),
        ],
    )(memory_embs, q2d, WQf, WALL, bQf)
    return vals[0, 2 * H:], bs[0, 0:2 * H:2], bi[0, 0:2 * H:2]


# speedup vs baseline: 1.0919x; 1.0919x over previous
import jax
import jax.numpy as jnp
from jax.experimental import pallas as pl
from jax.experimental.pallas import tpu as pltpu

D = 768
S = 8192
H = 10
W = 32
BLOCK_S = 4096

_SHIFTS = tuple(sorted({20 + j - 2 * min(j, 9) for j in range(12)}))
_DESTS = {s: tuple(j for j in range(12) if 20 + j - 2 * min(j, 9) == s)
          for s in _SHIFTS}


def _b16(x):
    return x.astype(jnp.bfloat16)


def _f32(x):
    return x.astype(jnp.float32)


def _head_to_val_lanes(x):
    lane = jax.lax.broadcasted_iota(jnp.int32, x.shape, 1)
    out = x
    for s in _SHIFTS:
        rolled = pltpu.roll(x, s, 1)
        dmask = jnp.zeros(x.shape, dtype=jnp.int32)
        for j in _DESTS[s]:
            dmask = jnp.maximum(dmask, (lane == 20 + j).astype(jnp.int32))
        out = jnp.where(dmask > 0, rolled, out)
    return out


def _body(mem_ref, q2d_ref, wqf_ref, wall_ref, bqf_ref,
          vals_ref, bs_ref, bi_ref,
          qm_s, m_s, idx_s, v_s):
    step = pl.program_id(0)
    nsteps = pl.num_programs(0)

    @pl.when(step == 0)
    def _init():
        qrow = jax.lax.dot_general(
            _b16(q2d_ref[:]), _b16(wqf_ref[:]), (((1,), (1,)), ((), ())),
            preferred_element_type=jnp.float32) + bqf_ref[:]
        qrow32 = _f32(_b16(jnp.concatenate(
            [qrow, jnp.zeros((1, W - 2 * H), jnp.float32)], axis=1)))
        qfull = jnp.broadcast_to(qrow32, (W, W))
        rollc = pltpu.roll(qfull, W - 1, 1)
        rr = jax.lax.broadcasted_iota(jnp.int32, (W, W), 0)
        cc = jax.lax.broadcasted_iota(jnp.int32, (W, W), 1)
        head = rr < 2 * H
        even_diag = jnp.logical_and(jnp.logical_and(rr == cc, rr % 2 == 0), head)
        odd_sub = jnp.logical_and(jnp.logical_and(cc == rr - 1, rr % 2 == 1), head)
        qm_s[:] = _b16(jnp.where(even_diag, qfull, 0.0)
                       + jnp.where(odd_sub, rollc, 0.0))
        m_s[:] = jnp.full((1, W), -jnp.inf, dtype=jnp.float32)
        idx_s[:] = jnp.zeros((1, W), dtype=jnp.int32)
        v_s[:] = jnp.zeros((1, W), dtype=jnp.float32)

    scat = jax.lax.dot_general(mem_ref[:], wall_ref[:], (((1,), (1,)), ((), ())),
                               preferred_element_type=jnp.float32)
    scores = jax.lax.dot_general(
        scat, _f32(qm_s[:]), (((1,), (0,)), ((), ())),
        preferred_element_type=jnp.float32)

    m = jnp.max(scores, axis=0, keepdims=True)
    ii = jax.lax.broadcasted_iota(jnp.int32, scores.shape, 0)
    li = jnp.min(jnp.where(scores == m, ii, BLOCK_S), axis=0, keepdims=True)
    li_all = _head_to_val_lanes(li)
    sel = jnp.where(ii == li_all, scat, 0.0)
    v_cand = jnp.sum(sel, axis=0, keepdims=True)

    upd = m > m_s[:]
    updv = _head_to_val_lanes(upd.astype(jnp.int32)) > 0
    m_s[:] = jnp.where(upd, m, m_s[:])
    idx_s[:] = jnp.where(upd, li + step * BLOCK_S, idx_s[:])
    v_s[:] = jnp.where(updv, v_cand, v_s[:])

    @pl.when(step == nsteps - 1)
    def _fin():
        vals_ref[:] = v_s[:]
        bs_ref[:] = m_s[:]
        bi_ref[:] = idx_s[:]


def kernel(query_emb, memory_embs, WQ, bQ, WK, WV_small, WV_call):
    q2d = query_emb.reshape(1, D)
    WALL = jnp.concatenate(
        [WK.reshape(2 * H, D), WV_small.reshape(9, D), WV_call], axis=0)
    WQf = WQ.reshape(2 * H, D)
    bQf = bQ.reshape(1, 2 * H)

    nsteps = S // BLOCK_S
    full = lambda shape: pl.BlockSpec(shape, lambda i: (0, 0))
    vals, bs, bi = pl.pallas_call(
        _body,
        grid=(nsteps,),
        in_specs=[
            pl.BlockSpec((BLOCK_S, D), lambda i: (i, 0)),
            full((1, D)), full((2 * H, D)), full((W, D)), full((1, 2 * H)),
        ],
        out_specs=[full((1, W)), full((1, W)), full((1, W))],
        out_shape=[
            jax.ShapeDtypeStruct((1, W), jnp.float32),
            jax.ShapeDtypeStruct((1, W), jnp.float32),
            jax.ShapeDtypeStruct((1, W), jnp.int32),
        ],
        scratch_shapes=[
            pltpu.VMEM((W, W), jnp.bfloat16),
            pltpu.VMEM((1, W), jnp.float32),
            pltpu.VMEM((1, W), jnp.int32),
            pltpu.VMEM((1, W), jnp.float32),
        ],
    )(memory_embs, q2d, WQf, WALL, bQf)
    return vals[0, 2 * H:], bs[0, 0:2 * H:2], bi[0, 0:2 * H:2]
